# baseline (speedup 1.0000x reference)
import functools

import jax
import jax.numpy as jnp
from jax import lax
from jax.experimental import pallas as pl
from jax.experimental.pallas import tpu as pltpu
from jax.experimental.pallas import tpu_sc as plsc

TILE = 1024
LANES = 16


def _routing_sc(B, S, E, D, ri, wrows):
    mesh = plsc.VectorSubcoreMesh(core_axis_name="c", subcore_axis_name="s")
    nchunks = D // LANES

    @functools.partial(
        pl.kernel,
        mesh=mesh,
        out_type=jax.ShapeDtypeStruct((B, D), jnp.float32),
        scratch_types=[
            pltpu.VMEM((S,), jnp.int32),
            pltpu.VMEM((E, D), jnp.float32),
            pltpu.VMEM((D,), jnp.float32),
            pltpu.VMEM((LANES,), jnp.float32),
        ],
        compiler_params=pltpu.CompilerParams(needs_layout_passes=False),
    )
    def sc_kernel(ri_hbm, w_hbm, weff_hbm, idx_v, w_v, weff_v, pres_v):
        wid = lax.axis_index("s") * 2 + lax.axis_index("c")

        @pl.when(wid < B)
        def _():
            pltpu.sync_copy(ri_hbm.at[wid], idx_v)
            pltpu.sync_copy(w_hbm.at[wid], w_v)

            ones = jnp.full((LANES,), 1.0, jnp.float32)
            pres_v[...] = jnp.zeros((LANES,), jnp.float32)

            def presence_step(k, carry):
                v = idx_v[pl.ds(k * LANES, LANES)]
                plsc.store_scatter(pres_v, [v + 1], ones)
                return carry

            lax.fori_loop(0, S // LANES, presence_step, 0)

            splats = [
                plsc.load_gather(pres_v, [jnp.full((LANES,), e + 1, jnp.int32)])
                for e in range(E)
            ]

            def row_step(k, carry):
                acc = jnp.zeros((LANES,), jnp.float32)
                for e in range(E):
                    acc = acc + splats[e] * w_v[e, pl.ds(k * LANES, LANES)]
                weff_v[pl.ds(k * LANES, LANES)] = acc
                return carry

            lax.fori_loop(0, nchunks, row_step, 0)
            pltpu.sync_copy(weff_v, weff_hbm.at[wid])

    return sc_kernel(ri, wrows)


def _moe_body(rw_ref, weff_ref, x_ref, y_ref):
    t = lax.dot_general(weff_ref[0], x_ref[0], (((1,), (1,)), ((), ())),
                        preferred_element_type=jnp.float32)
    y_ref[0] = rw_ref[0, 0][:, None] * t


def kernel(x, routing_weights, routing_indices, weights):
    B, S, D = x.shape
    E = weights.shape[0]
    rw = routing_weights.reshape(B, 1, S)
    ri = routing_indices.reshape(B, S)
    wrows = jnp.transpose(weights[:, :B, :], (1, 0, 2))

    weff = _routing_sc(B, S, E, D, ri, wrows).reshape(B, 1, D)

    ntiles = S // TILE
    return pl.pallas_call(
        _moe_body,
        grid=(B, ntiles),
        in_specs=[
            pl.BlockSpec((1, 1, S), lambda j, t: (j, 0, 0)),
            pl.BlockSpec((1, 1, D), lambda j, t: (j, 0, 0)),
            pl.BlockSpec((1, TILE, D), lambda j, t: (j, t, 0)),
        ],
        out_specs=pl.BlockSpec((1, S, TILE), lambda j, t: (j, 0, t)),
        out_shape=jax.ShapeDtypeStruct((B, S, S), x.dtype),
    )(rw, weff, x)


# speedup vs baseline: 1.0815x; 1.0815x over previous
import functools

import jax
import jax.numpy as jnp
from jax import lax
from jax.experimental import pallas as pl
from jax.experimental.pallas import tpu as pltpu
from jax.experimental.pallas import tpu_sc as plsc

TILE = 1024
LANES = 16


def _presence_sc(B, S, ri):
    mesh = plsc.VectorSubcoreMesh(core_axis_name="c", subcore_axis_name="s")

    @functools.partial(
        pl.kernel,
        mesh=mesh,
        out_type=jax.ShapeDtypeStruct((B, LANES), jnp.float32),
        scratch_types=[
            pltpu.VMEM((S,), jnp.int32),
            pltpu.VMEM((LANES,), jnp.float32),
            pltpu.VMEM((LANES,), jnp.float32),
        ],
        compiler_params=pltpu.CompilerParams(needs_layout_passes=False),
    )
    def sc_kernel(ri_hbm, pres_hbm, idx_v, tab_v, un_v):
        wid = lax.axis_index("s") * 2 + lax.axis_index("c")

        @pl.when(wid < B)
        def _():
            pltpu.sync_copy(ri_hbm.at[wid], idx_v)
            ones = jnp.full((LANES,), 1.0, jnp.float32)
            tab_v[...] = jnp.zeros((LANES,), jnp.float32)

            def presence_step(k, carry):
                v = idx_v[pl.ds(k * LANES, LANES)]
                plsc.store_scatter(tab_v, [v + 1], ones)
                return carry

            lax.fori_loop(0, S // LANES, presence_step, 0)

            iv = lax.iota(jnp.int32, LANES)
            un_v[...] = plsc.load_gather(tab_v, [jnp.minimum(iv + 1, LANES - 1)])
            pltpu.sync_copy(un_v, pres_hbm.at[wid])

    return sc_kernel(ri)


def _moe_body(pres_ref, rw_ref, w_ref, x_ref, y_ref, weff_ref):
    E = w_ref.shape[1]

    @pl.when(pl.program_id(1) == 0)
    def _():
        maskf = pres_ref[0][:, :E]
        weff_ref[...] = jnp.dot(maskf, w_ref[0],
                                preferred_element_type=jnp.float32)

    t = lax.dot_general(weff_ref[...], x_ref[0], (((1,), (1,)), ((), ())),
                        preferred_element_type=jnp.float32)

    y_ref[0] = rw_ref[0, 0][:, None] * t


def kernel(x, routing_weights, routing_indices, weights):
    B, S, D = x.shape
    E = weights.shape[0]
    rw = routing_weights.reshape(B, 1, S)
    ri = routing_indices.reshape(B, S)
    wrows = jnp.transpose(weights[:, :B, :], (1, 0, 2))

    pres = _presence_sc(B, S, ri).reshape(B, 1, LANES)

    ntiles = S // TILE
    return pl.pallas_call(
        _moe_body,
        grid=(B, ntiles),
        in_specs=[
            pl.BlockSpec((1, 1, LANES), lambda j, t: (j, 0, 0)),
            pl.BlockSpec((1, 1, S), lambda j, t: (j, 0, 0)),
            pl.BlockSpec((1, E, D), lambda j, t: (j, 0, 0)),
            pl.BlockSpec((1, TILE, D), lambda j, t: (j, t, 0)),
        ],
        out_specs=pl.BlockSpec((1, S, TILE), lambda j, t: (j, 0, t)),
        out_shape=jax.ShapeDtypeStruct((B, S, S), x.dtype),
        scratch_shapes=[pltpu.VMEM((1, D), jnp.float32)],
    )(pres, rw, wrows, x)


# speedup vs baseline: 1.1195x; 1.0351x over previous
import functools

import jax
import jax.numpy as jnp
from jax import lax
from jax.experimental import pallas as pl
from jax.experimental.pallas import tpu as pltpu
from jax.experimental.pallas import tpu_sc as plsc

TILE = 1024
LANES = 16


def _presence_sc(B, S, ri):
    mesh = plsc.VectorSubcoreMesh(core_axis_name="c", subcore_axis_name="s")

    @functools.partial(
        pl.kernel,
        mesh=mesh,
        out_type=jax.ShapeDtypeStruct((B, LANES), jnp.float32),
        scratch_types=[
            pltpu.VMEM((S,), jnp.int32),
            pltpu.VMEM((LANES,), jnp.float32),
            pltpu.VMEM((LANES,), jnp.float32),
        ],
        compiler_params=pltpu.CompilerParams(needs_layout_passes=False),
    )
    def sc_kernel(ri_hbm, pres_hbm, idx_v, tab_v, un_v):
        wid = lax.axis_index("s") * 2 + lax.axis_index("c")

        @pl.when(wid < B)
        def _():
            pltpu.sync_copy(ri_hbm.at[wid], idx_v)
            ones = jnp.full((LANES,), 1.0, jnp.float32)
            tab_v[...] = jnp.zeros((LANES,), jnp.float32)

            def presence_step(k, carry):
                v = idx_v[pl.ds(k * LANES, LANES)]
                plsc.store_scatter(tab_v, [v + 1], ones)
                return carry

            lax.fori_loop(0, S // LANES, presence_step, 0)

            iv = lax.iota(jnp.int32, LANES)
            un_v[...] = plsc.load_gather(tab_v, [jnp.minimum(iv + 1, LANES - 1)])
            pltpu.sync_copy(un_v, pres_hbm.at[wid])

    return sc_kernel(ri)


def _matvec_body(w_ref, x_ref, t_ref):
    t_ref[0] = lax.dot_general(w_ref[0], x_ref[0], (((1,), (1,)), ((), ())),
                               preferred_element_type=jnp.float32)


def _outer_body(pres_ref, rw_ref, tall_ref, y_ref, teff_ref):
    E = tall_ref.shape[1]

    @pl.when(pl.program_id(1) == 0)
    def _():
        maskf = pres_ref[0][:, :E]
        teff_ref[...] = jnp.sum(tall_ref[0] * maskf.reshape(E, 1),
                                axis=0, keepdims=True)

    tile = pl.program_id(1)
    t = teff_ref[:, pl.ds(tile * TILE, TILE)]
    y_ref[0] = rw_ref[0, 0][:, None] * t


def kernel(x, routing_weights, routing_indices, weights):
    B, S, D = x.shape
    E = weights.shape[0]
    rw = routing_weights.reshape(B, 1, S)
    ri = routing_indices.reshape(B, S)
    wrows = jnp.transpose(weights[:, :B, :], (1, 0, 2))

    pres = _presence_sc(B, S, ri).reshape(B, 1, LANES)

    ntiles = S // TILE
    t_all = pl.pallas_call(
        _matvec_body,
        grid=(B, ntiles),
        in_specs=[
            pl.BlockSpec((1, E, D), lambda j, t: (j, 0, 0)),
            pl.BlockSpec((1, TILE, D), lambda j, t: (j, t, 0)),
        ],
        out_specs=pl.BlockSpec((1, E, TILE), lambda j, t: (j, 0, t)),
        out_shape=jax.ShapeDtypeStruct((B, E, S), jnp.float32),
    )(wrows, x)

    return pl.pallas_call(
        _outer_body,
        grid=(B, ntiles),
        in_specs=[
            pl.BlockSpec((1, 1, LANES), lambda j, t: (j, 0, 0)),
            pl.BlockSpec((1, 1, S), lambda j, t: (j, 0, 0)),
            pl.BlockSpec((1, E, S), lambda j, t: (j, 0, 0)),
        ],
        out_specs=pl.BlockSpec((1, S, TILE), lambda j, t: (j, 0, t)),
        out_shape=jax.ShapeDtypeStruct((B, S, S), x.dtype),
        scratch_shapes=[pltpu.VMEM((1, S), jnp.float32)],
    )(pres, rw, t_all)
